# baseline (speedup 1.0000x reference)
import math
import functools

import jax
import jax.numpy as jnp
import numpy as np
from jax import lax
from jax.experimental import pallas as pl
from jax.experimental.pallas import tpu as pltpu
from jax.experimental.pallas import tpu_sc as plsc

SEQ = 200
BATCH = 4096
EMB_DIM = 64
MAX_LEN = 200
VOCAB = 1000000
B_TOTAL = SEQ * BATCH
NC, NS = 2, 16
NW = NC * NS
CHUNK = 256
SUB = 128
IDX_ROWS = CHUNK // SUB
CHUNKS_PER_S = BATCH // CHUNK
HALF_B = BATCH // 2
SCALE = math.sqrt(EMB_DIM)
NSL = EMB_DIM // 16

NSLICE = 5
SEQ_S = SEQ // NSLICE
CHUNKS_S = SEQ_S * CHUNKS_PER_S
N_CHUNKS = CHUNKS_S // NW
ROWS_S = SEQ_S * HALF_B


def _pe_table():
    position = np.arange(0, MAX_LEN, dtype=np.float32)[:, None]
    div_term = np.exp(
        np.arange(0, EMB_DIM, 2, dtype=np.float32) * -(math.log(10000.0) / EMB_DIM)
    )
    pe = np.zeros((MAX_LEN, EMB_DIM), dtype=np.float32)
    pe[:, 0::2] = np.sin(position * div_term)
    pe[:, 1::2] = np.cos(position * div_term)
    return pe.reshape(-1)


_PE_FLAT = _pe_table()

_mesh = plsc.VectorSubcoreMesh(
    core_axis_name="c", subcore_axis_name="s", num_cores=NC, num_subcores=NS
)

_BV = 7936
_BH = _BV // 2
_NBLK = (VOCAB + _BV - 1) // _BV


def _pack_body(wt_ref, out_ref):
    xt = wt_ref[...].T
    out_ref[:, 0:EMB_DIM] = xt[0:_BH]
    out_ref[:, EMB_DIM : 2 * EMB_DIM] = xt[_BH:_BV]


_pack_table = pl.pallas_call(
    _pack_body,
    grid=(_NBLK,),
    in_specs=[pl.BlockSpec((EMB_DIM, _BV), lambda i: (0, i))],
    out_specs=pl.BlockSpec((_BH, 2 * EMB_DIM), lambda i: (i, 0)),
    out_shape=jax.ShapeDtypeStruct((_NBLK * _BH, 2 * EMB_DIM), jnp.float32),
)


def _make_emb_slice(slice_idx):
    sbase = slice_idx * CHUNKS_S

    @functools.partial(
        pl.kernel,
        out_type=jax.ShapeDtypeStruct((ROWS_S, 2 * EMB_DIM), jnp.float32),
        mesh=_mesh,
        scratch_types=[
            pltpu.VMEM((N_CHUNKS * IDX_ROWS, SUB), jnp.int32),
            pltpu.VMEM((CHUNK, EMB_DIM), jnp.float32),
            pltpu.VMEM((CHUNK, EMB_DIM), jnp.float32),
            pltpu.VMEM((MAX_LEN * EMB_DIM,), jnp.float32),
            pltpu.SemaphoreType.DMA,
            pltpu.SemaphoreType.DMA,
            pltpu.SemaphoreType.DMA,
            pltpu.SemaphoreType.DMA,
        ],
        compiler_params=pltpu.CompilerParams(use_tc_tiling_on_sc=False),
    )
    def _emb_slice(table_hbm, idx_hbm, pe_hbm, out_hbm, idx_v, buf0, buf1,
                   pe_v, sg0, sg1, so0, so1):
        wid = lax.axis_index("s") * NC + lax.axis_index("c")
        chunk0 = wid * N_CHUNKS

        pltpu.sync_copy(pe_hbm, pe_v)
        pltpu.sync_copy(
            idx_hbm.at[pl.ds((sbase + chunk0) * IDX_ROWS, N_CHUNKS * IDX_ROWS)],
            idx_v,
        )

        def fire_gather(c, buf, sem):
            for u in range(IDX_ROWS):
                pltpu.async_copy(
                    table_hbm.at[idx_v.at[c * IDX_ROWS + u]],
                    buf.at[pl.ds(u * SUB, SUB)],
                    sem,
                )

        def wait_gather(buf, sem):
            for u in range(IDX_ROWS):
                pltpu.make_async_copy(
                    table_hbm.at[idx_v.at[0]], buf.at[pl.ds(u * SUB, SUB)], sem
                ).wait()

        def out_slice(c):
            sc = chunk0 + c
            s_pos = sc // CHUNKS_PER_S
            b0 = (sc % CHUNKS_PER_S) * CHUNK
            half = b0 // HALF_B
            jrow = s_pos * HALF_B + (b0 % HALF_B)
            return out_hbm.at[pl.ds(jrow, CHUNK), pl.ds(half * EMB_DIM, EMB_DIM)]

        def wait_out(buf, sem):
            pltpu.make_async_copy(buf, out_slice(0), sem).wait()

        def compute(c, buf):
            pe_off = ((sbase + chunk0 + c) // CHUNKS_PER_S) * EMB_DIM
            pes = [pe_v[pl.ds(pe_off + k * 16, 16)] for k in range(NSL)]

            def row_body(i, carry):
                for j in range(4):
                    r = i * 4 + j
                    for k in range(NSL):
                        sl = pl.ds(k * 16, 16)
                        buf[r, sl] = buf[r, sl] * SCALE + pes[k]
                return carry

            lax.fori_loop(0, CHUNK // 4, row_body, 0)

        def fire_out(c, buf, sem):
            pltpu.async_copy(buf, out_slice(c), sem)

        fire_gather(0, buf0, sg0)
        fire_gather(1, buf1, sg1)
        wait_gather(buf0, sg0)
        compute(0, buf0)
        fire_out(0, buf0, so0)
        wait_out(buf0, so0)
        fire_gather(2, buf0, sg0)
        wait_gather(buf1, sg1)
        compute(1, buf1)
        fire_out(1, buf1, so1)

        def pipe_body(t, carry):
            g0 = 2 * t
            wait_out(buf1, so1)
            fire_gather(g0 + 1, buf1, sg1)
            wait_gather(buf0, sg0)
            compute(g0, buf0)
            fire_out(g0, buf0, so0)
            wait_out(buf0, so0)
            fire_gather(g0 + 2, buf0, sg0)
            wait_gather(buf1, sg1)
            compute(g0 + 1, buf1)
            fire_out(g0 + 1, buf1, so1)
            return carry

        lax.fori_loop(1, N_CHUNKS // 2 - 1, pipe_body, 0)

        g0 = N_CHUNKS - 2
        wait_out(buf1, so1)
        fire_gather(g0 + 1, buf1, sg1)
        wait_gather(buf0, sg0)
        compute(g0, buf0)
        fire_out(g0, buf0, so0)
        wait_gather(buf1, sg1)
        compute(g0 + 1, buf1)
        fire_out(g0 + 1, buf1, so1)
        wait_out(buf0, so0)
        wait_out(buf1, so1)

    return _emb_slice


_emb_slices = [_make_emb_slice(i) for i in range(NSLICE)]


def _finish_first_body(o2_ref, out_ref):
    x = o2_ref[...]
    out_ref[0, :, 0:HALF_B] = x[:, 0:EMB_DIM].T
    out_ref[0, :, HALF_B:BATCH] = x[:, EMB_DIM : 2 * EMB_DIM].T


def _finish_chain_body(o2_ref, prev_ref, out_ref):
    del prev_ref
    _finish_first_body(o2_ref, out_ref)


def _make_finish(slice_idx):
    soff = slice_idx * SEQ_S
    out_spec = pl.BlockSpec((1, EMB_DIM, BATCH), lambda s: (soff + s, 0, 0))
    in_spec = pl.BlockSpec((HALF_B, 2 * EMB_DIM), lambda s: (s, 0))
    out_shape = jax.ShapeDtypeStruct((SEQ, EMB_DIM, BATCH), jnp.float32)
    if slice_idx == 0:
        return pl.pallas_call(
            _finish_first_body,
            grid=(SEQ_S,),
            in_specs=[in_spec],
            out_specs=out_spec,
            out_shape=out_shape,
        )
    return pl.pallas_call(
        _finish_chain_body,
        grid=(SEQ_S,),
        in_specs=[in_spec, pl.BlockSpec(memory_space=pl.ANY)],
        out_specs=out_spec,
        out_shape=out_shape,
        input_output_aliases={1: 0},
    )


_finishes = [_make_finish(i) for i in range(NSLICE)]


def kernel(x_val, W):
    v = x_val.astype(jnp.int32)
    blk = v // _BV
    q = v % _BV
    lin = blk * _BV + 2 * (q % _BH) + (q // _BH)
    idx2d = lin.reshape(B_TOTAL // SUB, SUB)
    pe_flat = jnp.asarray(_PE_FLAT)
    w_lin = _pack_table(W.T).reshape(_NBLK * _BV, EMB_DIM)
    o2s = [emb(w_lin, idx2d, pe_flat) for emb in _emb_slices]
    out_t = _finishes[0](o2s[0])
    for i in range(1, NSLICE):
        out_t = _finishes[i](o2s[i], out_t)
    return jnp.swapaxes(out_t, 1, 2)


# speedup vs baseline: 1.1065x; 1.1065x over previous
import math
import functools

import jax
import jax.numpy as jnp
import numpy as np
from jax import lax
from jax.experimental import pallas as pl
from jax.experimental.pallas import tpu as pltpu
from jax.experimental.pallas import tpu_sc as plsc

SEQ = 200
BATCH = 4096
EMB_DIM = 64
MAX_LEN = 200
VOCAB = 1000000
B_TOTAL = SEQ * BATCH
NC, NS = 2, 16
NW = NC * NS
CHUNK = 256
SUB = 128
IDX_ROWS = CHUNK // SUB
CHUNKS_PER_S = BATCH // CHUNK
HALF_B = BATCH // 2
SCALE = math.sqrt(EMB_DIM)
NSL = EMB_DIM // 16

NSLICE = 5
SEQ_S = SEQ // NSLICE
CHUNKS_S = SEQ_S * CHUNKS_PER_S
N_CHUNKS = CHUNKS_S // NW
ROWS_S = SEQ_S * HALF_B


def _pe_table():
    position = np.arange(0, MAX_LEN, dtype=np.float32)[:, None]
    div_term = np.exp(
        np.arange(0, EMB_DIM, 2, dtype=np.float32) * -(math.log(10000.0) / EMB_DIM)
    )
    pe = np.zeros((MAX_LEN, EMB_DIM), dtype=np.float32)
    pe[:, 0::2] = np.sin(position * div_term)
    pe[:, 1::2] = np.cos(position * div_term)
    return pe.reshape(-1)


_PE_FLAT = _pe_table()

_mesh = plsc.VectorSubcoreMesh(
    core_axis_name="c", subcore_axis_name="s", num_cores=NC, num_subcores=NS
)

_BV = 7936
_BH = _BV // 2
_NBLK = (VOCAB + _BV - 1) // _BV


def _pack_body(wt_ref, out_ref):
    xt = wt_ref[...].T
    out_ref[:, 0:EMB_DIM] = xt[0:_BH]
    out_ref[:, EMB_DIM : 2 * EMB_DIM] = xt[_BH:_BV]


_pack_table = pl.pallas_call(
    _pack_body,
    grid=(_NBLK,),
    in_specs=[pl.BlockSpec((EMB_DIM, _BV), lambda i: (0, i))],
    out_specs=pl.BlockSpec((_BH, 2 * EMB_DIM), lambda i: (i, 0)),
    out_shape=jax.ShapeDtypeStruct((_NBLK * _BH, 2 * EMB_DIM), jnp.float32),
)


def _make_emb_slice(slice_idx):
    sbase = slice_idx * CHUNKS_S

    @functools.partial(
        pl.kernel,
        out_type=jax.ShapeDtypeStruct((ROWS_S, 2 * EMB_DIM), jnp.float32),
        mesh=_mesh,
        scratch_types=[
            pltpu.VMEM((N_CHUNKS * IDX_ROWS, SUB), jnp.int32),
            pltpu.VMEM((CHUNK, EMB_DIM), jnp.float32),
            pltpu.VMEM((CHUNK, EMB_DIM), jnp.float32),
            pltpu.VMEM((MAX_LEN * EMB_DIM,), jnp.float32),
            pltpu.SemaphoreType.DMA,
            pltpu.SemaphoreType.DMA,
            pltpu.SemaphoreType.DMA,
            pltpu.SemaphoreType.DMA,
        ],
        compiler_params=pltpu.CompilerParams(use_tc_tiling_on_sc=False),
    )
    def _emb_slice(table_hbm, idx_hbm, pe_hbm, out_hbm, idx_v, buf0, buf1,
                   pe_v, sg0, sg1, so0, so1):
        wid = lax.axis_index("s") * NC + lax.axis_index("c")
        chunk0 = wid * N_CHUNKS

        pltpu.sync_copy(pe_hbm, pe_v)
        pltpu.sync_copy(
            idx_hbm.at[pl.ds((sbase + chunk0) * IDX_ROWS, N_CHUNKS * IDX_ROWS)],
            idx_v,
        )

        def fire_gather(c, buf, sem):
            for u in range(IDX_ROWS):
                pltpu.async_copy(
                    table_hbm.at[idx_v.at[c * IDX_ROWS + u]],
                    buf.at[pl.ds(u * SUB, SUB)],
                    sem,
                )

        def wait_gather(buf, sem):
            for u in range(IDX_ROWS):
                pltpu.make_async_copy(
                    table_hbm.at[idx_v.at[0]], buf.at[pl.ds(u * SUB, SUB)], sem
                ).wait()

        def out_slice(c):
            sc = chunk0 + c
            s_pos = sc // CHUNKS_PER_S
            b0 = (sc % CHUNKS_PER_S) * CHUNK
            half = b0 // HALF_B
            jrow = s_pos * HALF_B + (b0 % HALF_B)
            return out_hbm.at[pl.ds(jrow, CHUNK), pl.ds(half * EMB_DIM, EMB_DIM)]

        def wait_out(buf, sem):
            pltpu.make_async_copy(buf, out_slice(0), sem).wait()

        def compute(c, buf):
            pe_off = ((sbase + chunk0 + c) // CHUNKS_PER_S) * EMB_DIM
            pes = [pe_v[pl.ds(pe_off + k * 16, 16)] for k in range(NSL)]

            def row_body(i, carry):
                for j in range(4):
                    r = i * 4 + j
                    for k in range(NSL):
                        sl = pl.ds(k * 16, 16)
                        buf[r, sl] = buf[r, sl] * SCALE + pes[k]
                return carry

            lax.fori_loop(0, CHUNK // 4, row_body, 0)

        def fire_out(c, buf, sem):
            pltpu.async_copy(buf, out_slice(c), sem)

        fire_gather(0, buf0, sg0)
        fire_gather(1, buf1, sg1)
        wait_gather(buf0, sg0)
        compute(0, buf0)
        fire_out(0, buf0, so0)
        wait_out(buf0, so0)
        fire_gather(2, buf0, sg0)
        wait_gather(buf1, sg1)
        compute(1, buf1)
        fire_out(1, buf1, so1)

        def pipe_body(t, carry):
            g0 = 2 * t
            wait_out(buf1, so1)
            fire_gather(g0 + 1, buf1, sg1)
            wait_gather(buf0, sg0)
            compute(g0, buf0)
            fire_out(g0, buf0, so0)
            wait_out(buf0, so0)
            fire_gather(g0 + 2, buf0, sg0)
            wait_gather(buf1, sg1)
            compute(g0 + 1, buf1)
            fire_out(g0 + 1, buf1, so1)
            return carry

        lax.fori_loop(1, N_CHUNKS // 2 - 1, pipe_body, 0)

        g0 = N_CHUNKS - 2
        wait_out(buf1, so1)
        fire_gather(g0 + 1, buf1, sg1)
        wait_gather(buf0, sg0)
        compute(g0, buf0)
        fire_out(g0, buf0, so0)
        wait_gather(buf1, sg1)
        compute(g0 + 1, buf1)
        fire_out(g0 + 1, buf1, so1)
        wait_out(buf0, so0)
        wait_out(buf1, so1)

    return _emb_slice


_emb_slices = [_make_emb_slice(i) for i in range(NSLICE)]


_FP = 4


def _finish_first_body(o2_ref, out_ref):
    y = o2_ref[...].T
    for p in range(_FP):
        lo, hi = p * HALF_B, (p + 1) * HALF_B
        out_ref[p, :, 0:HALF_B] = y[0:EMB_DIM, lo:hi]
        out_ref[p, :, HALF_B:BATCH] = y[EMB_DIM : 2 * EMB_DIM, lo:hi]


def _finish_chain_body(o2_ref, prev_ref, out_ref):
    del prev_ref
    _finish_first_body(o2_ref, out_ref)


def _make_finish(slice_idx):
    soff = slice_idx * SEQ_S // _FP
    out_spec = pl.BlockSpec((_FP, EMB_DIM, BATCH), lambda s: (soff + s, 0, 0))
    in_spec = pl.BlockSpec((_FP * HALF_B, 2 * EMB_DIM), lambda s: (s, 0))
    out_shape = jax.ShapeDtypeStruct((SEQ, EMB_DIM, BATCH), jnp.float32)
    if slice_idx == 0:
        return pl.pallas_call(
            _finish_first_body,
            grid=(SEQ_S // _FP,),
            in_specs=[in_spec],
            out_specs=out_spec,
            out_shape=out_shape,
        )
    return pl.pallas_call(
        _finish_chain_body,
        grid=(SEQ_S // _FP,),
        in_specs=[in_spec, pl.BlockSpec(memory_space=pl.ANY)],
        out_specs=out_spec,
        out_shape=out_shape,
        input_output_aliases={1: 0},
    )


_finishes = [_make_finish(i) for i in range(NSLICE)]


def kernel(x_val, W):
    v = x_val.astype(jnp.int32)
    blk = v // _BV
    q = v % _BV
    lin = blk * _BV + 2 * (q % _BH) + (q // _BH)
    idx2d = lin.reshape(B_TOTAL // SUB, SUB)
    pe_flat = jnp.asarray(_PE_FLAT)
    w_lin = _pack_table(W.T).reshape(_NBLK * _BV, EMB_DIM)
    o2s = [emb(w_lin, idx2d, pe_flat) for emb in _emb_slices]
    out_t = _finishes[0](o2s[0])
    for i in range(1, NSLICE):
        out_t = _finishes[i](o2s[i], out_t)
    return jnp.swapaxes(out_t, 1, 2)


# speedup vs baseline: 1.1072x; 1.0006x over previous
import math
import functools

import jax
import jax.numpy as jnp
import numpy as np
from jax import lax
from jax.experimental import pallas as pl
from jax.experimental.pallas import tpu as pltpu
from jax.experimental.pallas import tpu_sc as plsc

SEQ = 200
BATCH = 4096
EMB_DIM = 64
MAX_LEN = 200
VOCAB = 1000000
B_TOTAL = SEQ * BATCH
NC, NS = 2, 16
NW = NC * NS
CHUNK = 256
SUB = 128
IDX_ROWS = CHUNK // SUB
CHUNKS_PER_S = BATCH // CHUNK
HALF_B = BATCH // 2
SCALE = math.sqrt(EMB_DIM)
NSL = EMB_DIM // 16

NSLICE = 5
SEQ_S = SEQ // NSLICE
CHUNKS_S = SEQ_S * CHUNKS_PER_S
N_CHUNKS = CHUNKS_S // NW
ROWS_S = SEQ_S * HALF_B


def _pe_table():
    position = np.arange(0, MAX_LEN, dtype=np.float32)[:, None]
    div_term = np.exp(
        np.arange(0, EMB_DIM, 2, dtype=np.float32) * -(math.log(10000.0) / EMB_DIM)
    )
    pe = np.zeros((MAX_LEN, EMB_DIM), dtype=np.float32)
    pe[:, 0::2] = np.sin(position * div_term)
    pe[:, 1::2] = np.cos(position * div_term)
    return pe.reshape(-1)


_PE_FLAT = _pe_table()

_mesh = plsc.VectorSubcoreMesh(
    core_axis_name="c", subcore_axis_name="s", num_cores=NC, num_subcores=NS
)

_BV = 7936
_BH = _BV // 2
_NBLK = (VOCAB + _BV - 1) // _BV


def _pack_body(wt_ref, out_ref):
    xt = wt_ref[...].T
    out_ref[...] = jnp.concatenate([xt[0:_BH], xt[_BH:_BV]], axis=1)


_pack_table = pl.pallas_call(
    _pack_body,
    grid=(_NBLK,),
    in_specs=[pl.BlockSpec((EMB_DIM, _BV), lambda i: (0, i))],
    out_specs=pl.BlockSpec((_BH, 2 * EMB_DIM), lambda i: (i, 0)),
    out_shape=jax.ShapeDtypeStruct((_NBLK * _BH, 2 * EMB_DIM), jnp.float32),
)


def _make_emb_slice(slice_idx):
    sbase = slice_idx * CHUNKS_S

    @functools.partial(
        pl.kernel,
        out_type=jax.ShapeDtypeStruct((ROWS_S, 2 * EMB_DIM), jnp.float32),
        mesh=_mesh,
        scratch_types=[
            pltpu.VMEM((N_CHUNKS * IDX_ROWS, SUB), jnp.int32),
            pltpu.VMEM((CHUNK, EMB_DIM), jnp.float32),
            pltpu.VMEM((CHUNK, EMB_DIM), jnp.float32),
            pltpu.VMEM((MAX_LEN * EMB_DIM,), jnp.float32),
            pltpu.SemaphoreType.DMA,
            pltpu.SemaphoreType.DMA,
            pltpu.SemaphoreType.DMA,
            pltpu.SemaphoreType.DMA,
        ],
        compiler_params=pltpu.CompilerParams(use_tc_tiling_on_sc=False),
    )
    def _emb_slice(table_hbm, idx_hbm, pe_hbm, out_hbm, idx_v, buf0, buf1,
                   pe_v, sg0, sg1, so0, so1):
        wid = lax.axis_index("s") * NC + lax.axis_index("c")
        chunk0 = wid * N_CHUNKS

        pltpu.sync_copy(pe_hbm, pe_v)
        pltpu.sync_copy(
            idx_hbm.at[pl.ds((sbase + chunk0) * IDX_ROWS, N_CHUNKS * IDX_ROWS)],
            idx_v,
        )

        def fire_gather(c, buf, sem):
            for u in range(IDX_ROWS):
                pltpu.async_copy(
                    table_hbm.at[idx_v.at[c * IDX_ROWS + u]],
                    buf.at[pl.ds(u * SUB, SUB)],
                    sem,
                )

        def wait_gather(buf, sem):
            for u in range(IDX_ROWS):
                pltpu.make_async_copy(
                    table_hbm.at[idx_v.at[0]], buf.at[pl.ds(u * SUB, SUB)], sem
                ).wait()

        def out_slice(c):
            sc = chunk0 + c
            s_pos = sc // CHUNKS_PER_S
            b0 = (sc % CHUNKS_PER_S) * CHUNK
            half = b0 // HALF_B
            jrow = s_pos * HALF_B + (b0 % HALF_B)
            return out_hbm.at[pl.ds(jrow, CHUNK), pl.ds(half * EMB_DIM, EMB_DIM)]

        def wait_out(buf, sem):
            pltpu.make_async_copy(buf, out_slice(0), sem).wait()

        def compute(c, buf):
            pe_off = ((sbase + chunk0 + c) // CHUNKS_PER_S) * EMB_DIM
            pes = [pe_v[pl.ds(pe_off + k * 16, 16)] for k in range(NSL)]

            def row_body(i, carry):
                for j in range(4):
                    r = i * 4 + j
                    for k in range(NSL):
                        sl = pl.ds(k * 16, 16)
                        buf[r, sl] = buf[r, sl] * SCALE + pes[k]
                return carry

            lax.fori_loop(0, CHUNK // 4, row_body, 0)

        def fire_out(c, buf, sem):
            pltpu.async_copy(buf, out_slice(c), sem)

        fire_gather(0, buf0, sg0)
        fire_gather(1, buf1, sg1)
        wait_gather(buf0, sg0)
        compute(0, buf0)
        fire_out(0, buf0, so0)
        wait_out(buf0, so0)
        fire_gather(2, buf0, sg0)
        wait_gather(buf1, sg1)
        compute(1, buf1)
        fire_out(1, buf1, so1)

        def pipe_body(t, carry):
            g0 = 2 * t
            wait_out(buf1, so1)
            fire_gather(g0 + 1, buf1, sg1)
            wait_gather(buf0, sg0)
            compute(g0, buf0)
            fire_out(g0, buf0, so0)
            wait_out(buf0, so0)
            fire_gather(g0 + 2, buf0, sg0)
            wait_gather(buf1, sg1)
            compute(g0 + 1, buf1)
            fire_out(g0 + 1, buf1, so1)
            return carry

        lax.fori_loop(1, N_CHUNKS // 2 - 1, pipe_body, 0)

        g0 = N_CHUNKS - 2
        wait_out(buf1, so1)
        fire_gather(g0 + 1, buf1, sg1)
        wait_gather(buf0, sg0)
        compute(g0, buf0)
        fire_out(g0, buf0, so0)
        wait_gather(buf1, sg1)
        compute(g0 + 1, buf1)
        fire_out(g0 + 1, buf1, so1)
        wait_out(buf0, so0)
        wait_out(buf1, so1)

    return _emb_slice


_emb_slices = [_make_emb_slice(i) for i in range(NSLICE)]


_FP = 4


def _finish_first_body(o2_ref, out_ref):
    y = o2_ref[...].T
    for p in range(_FP):
        lo, hi = p * HALF_B, (p + 1) * HALF_B
        out_ref[p, :, 0:HALF_B] = y[0:EMB_DIM, lo:hi]
        out_ref[p, :, HALF_B:BATCH] = y[EMB_DIM : 2 * EMB_DIM, lo:hi]


def _finish_chain_body(o2_ref, prev_ref, out_ref):
    del prev_ref
    _finish_first_body(o2_ref, out_ref)


def _make_finish(slice_idx):
    soff = slice_idx * SEQ_S // _FP
    out_spec = pl.BlockSpec((_FP, EMB_DIM, BATCH), lambda s: (soff + s, 0, 0))
    in_spec = pl.BlockSpec((_FP * HALF_B, 2 * EMB_DIM), lambda s: (s, 0))
    out_shape = jax.ShapeDtypeStruct((SEQ, EMB_DIM, BATCH), jnp.float32)
    if slice_idx == 0:
        return pl.pallas_call(
            _finish_first_body,
            grid=(SEQ_S // _FP,),
            in_specs=[in_spec],
            out_specs=out_spec,
            out_shape=out_shape,
        )
    return pl.pallas_call(
        _finish_chain_body,
        grid=(SEQ_S // _FP,),
        in_specs=[in_spec, pl.BlockSpec(memory_space=pl.ANY)],
        out_specs=out_spec,
        out_shape=out_shape,
        input_output_aliases={1: 0},
    )


_finishes = [_make_finish(i) for i in range(NSLICE)]


def kernel(x_val, W):
    v = x_val.astype(jnp.int32)
    blk = v // _BV
    q = v % _BV
    lin = blk * _BV + 2 * (q % _BH) + (q // _BH)
    idx2d = lin.reshape(B_TOTAL // SUB, SUB)
    pe_flat = jnp.asarray(_PE_FLAT)
    w_lin = _pack_table(W.T).reshape(_NBLK * _BV, EMB_DIM)
    o2s = [emb(w_lin, idx2d, pe_flat) for emb in _emb_slices]
    out_t = _finishes[0](o2s[0])
    for i in range(1, NSLICE):
        out_t = _finishes[i](o2s[i], out_t)
    return jnp.swapaxes(out_t, 1, 2)
